# baseline (speedup 1.0000x reference)
import jax
from jax.experimental import pallas as pl
from jax.experimental.pallas import tpu as pltpu

_ROWS = 100000
_GRID = 4
_RBLK = _ROWS // _GRID
_MBLK = _ROWS // _GRID


def _copy_body(emb_in, masks_in, emb_out, masks_out):
    emb_out[...] = emb_in[...]
    masks_out[...] = masks_in[...]


def kernel(embeddings, blade_masks):
    emb_out, masks_out = pl.pallas_call(
        _copy_body,
        grid=(_GRID,),
        in_specs=[
            pl.BlockSpec((_RBLK, 128), lambda i: (i, 0)),
            pl.BlockSpec((8, _ROWS), lambda i: (0, 0)),
        ],
        out_specs=[
            pl.BlockSpec((_RBLK, 128), lambda i: (i, 0)),
            pl.BlockSpec((8, _ROWS), lambda i: (0, 0)),
        ],
        out_shape=[
            jax.ShapeDtypeStruct(embeddings.shape, embeddings.dtype),
            jax.ShapeDtypeStruct(blade_masks.shape, blade_masks.dtype),
        ],
    )(embeddings, blade_masks)
    return (emb_out, masks_out)


# speedup vs baseline: 1.0173x; 1.0173x over previous
import jax
from jax.experimental import pallas as pl
from jax.experimental.pallas import tpu as pltpu

_ROWS = 100000
_GRID = 5
_RBLK = _ROWS // _GRID
_MBLK = _ROWS // _GRID


def _copy_body(emb_in, masks_in, emb_out, masks_out):
    emb_out[...] = emb_in[...]
    masks_out[...] = masks_in[...]


def kernel(embeddings, blade_masks):
    emb_out, masks_out = pl.pallas_call(
        _copy_body,
        grid=(_GRID,),
        in_specs=[
            pl.BlockSpec((_RBLK, 128), lambda i: (i, 0)),
            pl.BlockSpec((8, _ROWS), lambda i: (0, 0)),
        ],
        out_specs=[
            pl.BlockSpec((_RBLK, 128), lambda i: (i, 0)),
            pl.BlockSpec((8, _ROWS), lambda i: (0, 0)),
        ],
        out_shape=[
            jax.ShapeDtypeStruct(embeddings.shape, embeddings.dtype),
            jax.ShapeDtypeStruct(blade_masks.shape, blade_masks.dtype),
        ],
    )(embeddings, blade_masks)
    return (emb_out, masks_out)


# speedup vs baseline: 1.0196x; 1.0023x over previous
import jax
from jax.experimental import pallas as pl

_ROWS = 100000
_GRID = 5
_RBLK = _ROWS // _GRID


def _copy_body(emb_in, masks_in, emb_out, masks_out):
    emb_out[...] = emb_in[...]
    masks_out[...] = masks_in[...]


def kernel(embeddings, blade_masks):
    emb_out, masks_out = pl.pallas_call(
        _copy_body,
        grid=(_GRID,),
        in_specs=[
            pl.BlockSpec((_RBLK, 128), lambda i: (i, 0)),
            pl.BlockSpec((8, _ROWS), lambda i: (0, 0)),
        ],
        out_specs=[
            pl.BlockSpec((_RBLK, 128), lambda i: (i, 0)),
            pl.BlockSpec((8, _ROWS), lambda i: (0, 0)),
        ],
        out_shape=[
            jax.ShapeDtypeStruct(embeddings.shape, embeddings.dtype),
            jax.ShapeDtypeStruct(blade_masks.shape, blade_masks.dtype),
        ],
    )(embeddings, blade_masks)
    return (emb_out, masks_out)
